# baseline (speedup 1.0000x reference)
import functools

import jax
import jax.numpy as jnp
from jax import lax
from jax.experimental import pallas as pl
from jax.experimental.pallas import tpu as pltpu
from jax.experimental.pallas import tpu_sc as plsc

B = 16384
V = 100000
D = 64
NNUM = 4
NCAT = 26
T = NNUM + NCAT
NC = 2
NS = 16
NW = NC * NS
RPW = B // NW
GCH = 128
NG = RPW // GCH
LANES = 16
DC = D // LANES

_MESH = plsc.VectorSubcoreMesh(
    core_axis_name="c", subcore_axis_name="s", num_cores=NC, num_subcores=NS
)


def _body(nums, cats, ws, bs, *rest):
    tbls = rest[:NCAT]
    out = rest[NCAT]
    (idx_v, nums_v, wb_v, rows_v, nrows_v, sems) = rest[NCAT + 1:]

    wid = lax.axis_index("s") * NC + lax.axis_index("c")
    base = wid * RPW

    pltpu.sync_copy(cats.at[wid], idx_v)
    pltpu.sync_copy(nums.at[wid], nums_v)
    pltpu.sync_copy(ws, wb_v.at[0])
    pltpu.sync_copy(bs, wb_v.at[1])

    def fire(f):
        p = f % 2
        return [
            pltpu.async_copy(
                tbls[f].at[idx_v.at[f, j]],
                rows_v.at[p, pl.ds(j * GCH, GCH)],
                sems.at[p],
            )
            for j in range(NG)
        ]

    in_flight = fire(0)

    for n in range(NNUM):
        w_regs = [wb_v[0, n, pl.ds(dc * LANES, LANES)] for dc in range(DC)]
        b_regs = [wb_v[1, n, pl.ds(dc * LANES, LANES)] for dc in range(DC)]

        def chunk_fn(rc, _, n=n, w_regs=w_regs, b_regs=b_regs):
            i0 = pl.multiple_of(rc * LANES, LANES)
            v16 = nums_v[n, pl.ds(i0, LANES)]
            nan16 = v16 != v16
            s16 = jnp.where(nan16, 0.0, v16)
            t16 = jnp.where(nan16, 0.0, 1.0)
            for k in range(LANES):
                s = s16[k]
                t = t16[k]
                for dc in range(DC):
                    nrows_v[i0 + k, pl.ds(dc * LANES, LANES)] = (
                        s * w_regs[dc] + t * b_regs[dc]
                    )
            return 0

        lax.fori_loop(0, RPW // LANES, chunk_fn, 0)
        pltpu.sync_copy(nrows_v, out.at[pl.ds(base, RPW), n])

    for f in range(NCAT):
        for d in in_flight:
            d.wait()
        if f + 1 < NCAT:
            in_flight = fire(f + 1)
        pltpu.sync_copy(rows_v.at[f % 2], out.at[pl.ds(base, RPW), NNUM + f])


_tokenize = pl.kernel(
    _body,
    out_type=jax.ShapeDtypeStruct((B, T, D), jnp.float32),
    mesh=_MESH,
    compiler_params=pltpu.CompilerParams(use_tc_tiling_on_sc=False),
    scratch_types=[
        pltpu.VMEM((NCAT, NG, GCH), jnp.int32),
        pltpu.VMEM((NNUM, RPW), jnp.float32),
        pltpu.VMEM((2, NNUM, D), jnp.float32),
        pltpu.VMEM((2, RPW, D), jnp.float32),
        pltpu.VMEM((RPW, D), jnp.float32),
        pltpu.SemaphoreType.DMA((2,)),
    ],
)


@jax.jit
def kernel(num_f0, num_f1, num_f2, num_f3, cat_f0, cat_f1, cat_f2, cat_f3, cat_f4, cat_f5, cat_f6, cat_f7, cat_f8, cat_f9, cat_f10, cat_f11, cat_f12, cat_f13, cat_f14, cat_f15, cat_f16, cat_f17, cat_f18, cat_f19, cat_f20, cat_f21, cat_f22, cat_f23, cat_f24, cat_f25, W_num_f0, W_num_f1, W_num_f2, W_num_f3, b_num_f0, b_num_f1, b_num_f2, b_num_f3, emb_cat_f0, emb_cat_f1, emb_cat_f2, emb_cat_f3, emb_cat_f4, emb_cat_f5, emb_cat_f6, emb_cat_f7, emb_cat_f8, emb_cat_f9, emb_cat_f10, emb_cat_f11, emb_cat_f12, emb_cat_f13, emb_cat_f14, emb_cat_f15, emb_cat_f16, emb_cat_f17, emb_cat_f18, emb_cat_f19, emb_cat_f20, emb_cat_f21, emb_cat_f22, emb_cat_f23, emb_cat_f24, emb_cat_f25):
    nums = [num_f0, num_f1, num_f2, num_f3]
    cats = [cat_f0, cat_f1, cat_f2, cat_f3, cat_f4, cat_f5, cat_f6, cat_f7,
            cat_f8, cat_f9, cat_f10, cat_f11, cat_f12, cat_f13, cat_f14,
            cat_f15, cat_f16, cat_f17, cat_f18, cat_f19, cat_f20, cat_f21,
            cat_f22, cat_f23, cat_f24, cat_f25]
    Ws = [W_num_f0, W_num_f1, W_num_f2, W_num_f3]
    bs = [b_num_f0, b_num_f1, b_num_f2, b_num_f3]
    tbls = [emb_cat_f0, emb_cat_f1, emb_cat_f2, emb_cat_f3, emb_cat_f4,
            emb_cat_f5, emb_cat_f6, emb_cat_f7, emb_cat_f8, emb_cat_f9,
            emb_cat_f10, emb_cat_f11, emb_cat_f12, emb_cat_f13, emb_cat_f14,
            emb_cat_f15, emb_cat_f16, emb_cat_f17, emb_cat_f18, emb_cat_f19,
            emb_cat_f20, emb_cat_f21, emb_cat_f22, emb_cat_f23, emb_cat_f24,
            emb_cat_f25]

    cats_w = (
        jnp.stack([c.astype(jnp.int32) for c in cats])
        .reshape(NCAT, NW, NG, GCH)
        .transpose(1, 0, 2, 3)
    )
    nums_w = jnp.stack(nums).reshape(NNUM, NW, RPW).transpose(1, 0, 2)
    ws = jnp.stack(Ws)
    bs_ = jnp.stack(bs)
    return _tokenize(nums_w, cats_w, ws, bs_, *tbls)
